# baseline (speedup 1.0000x reference)
import functools

import jax
import jax.numpy as jnp
from jax import lax
from jax.experimental import pallas as pl
from jax.experimental.pallas import tpu as pltpu
from jax.experimental.pallas import tpu_sc as plsc

B, MB, ML = 1024, 64, 64
V, DC, DM = 100000, 64, 256
NBARS = B * MB
L = 16
NLC = DC // L

NC, NS = 2, 16
NW = NC * NS
BARS_PER_TILE = NBARS // NW
CB = 64
CHUNKS = BARS_PER_TILE // CB


def _sc_pool_body(idx_hbm, msk_hbm, table_hbm, pooled_hbm,
                  idx_v, msk_v, idxe_v, rows_v, outc_v, sem):
    wid = lax.axis_index("s") * NC + lax.axis_index("c")
    tile_base = wid * BARS_PER_TILE

    def chunk_body(ci, _):
        row0 = tile_base + ci * CB
        pltpu.sync_copy(idx_hbm.at[pl.ds(row0, CB)], idx_v)
        pltpu.sync_copy(msk_hbm.at[pl.ds(row0, CB)], msk_v)

        def bar_body(bi, _):
            for c in range(NLC):
                m = msk_v[bi, pl.ds(c * L, L)]
                idxe_v[pl.ds(c * L, L)] = idx_v[bi, pl.ds(c * L, L)] * m
            pltpu.async_copy(table_hbm.at[idxe_v], rows_v, sem).wait()
            acc = [jnp.zeros((L,), jnp.float32) for _ in range(NLC)]
            for j in range(ML):
                for c in range(NLC):
                    acc[c] = acc[c] + rows_v[j, pl.ds(c * L, L)]
            for c in range(NLC):
                outc_v[bi, pl.ds(c * L, L)] = acc[c]
            return 0

        lax.fori_loop(0, CB, bar_body, 0)
        pltpu.sync_copy(outc_v, pooled_hbm.at[pl.ds(row0, CB)])
        return 0

    lax.fori_loop(0, CHUNKS, chunk_body, 0)


_sc_pool = functools.partial(
    pl.kernel,
    out_type=jax.ShapeDtypeStruct((NBARS, DC), jnp.float32),
    mesh=plsc.VectorSubcoreMesh(core_axis_name="c", subcore_axis_name="s"),
    scratch_types=[
        pltpu.VMEM((CB, ML), jnp.int32),
        pltpu.VMEM((CB, ML), jnp.int32),
        pltpu.VMEM((ML,), jnp.int32),
        pltpu.VMEM((ML, DC), jnp.float32),
        pltpu.VMEM((CB, DC), jnp.float32),
        pltpu.SemaphoreType.DMA,
    ],
    compiler_params=pltpu.CompilerParams(use_tc_tiling_on_sc=False),
)(_sc_pool_body)


def _tc_body(x_ref, m_ref, w_ref, b_ref, pos_ref, g_ref, bb_ref, o_ref):
    x = x_ref[...]
    cnt = jnp.sum(m_ref[...].astype(jnp.float32), axis=-1, keepdims=True)
    rinv = 1.0 / jnp.maximum(cnt, 1.0)
    y = jnp.dot(x, w_ref[...], preferred_element_type=jnp.float32)
    y = y * rinv + b_ref[...]
    r = x.shape[0] // MB
    y = (y.reshape(r, MB, DM) + pos_ref[...][None]).reshape(r * MB, DM)
    mu = jnp.mean(y, axis=-1, keepdims=True)
    d = y - mu
    var = jnp.mean(d * d, axis=-1, keepdims=True)
    o_ref[...] = d * lax.rsqrt(var + 1e-5) * g_ref[...] + bb_ref[...]


def kernel(bar_indices, char_mask, bar_mask, char_table, W, b, pos_table,
           gamma, beta):
    idx = bar_indices.astype(jnp.int32).reshape(NBARS, ML)
    msk = char_mask.astype(jnp.int32).reshape(NBARS, ML)

    pooled = _sc_pool(idx, msk, char_table)

    R = 512
    out = pl.pallas_call(
        _tc_body,
        grid=(NBARS // R,),
        in_specs=[
            pl.BlockSpec((R, DC), lambda i: (i, 0)),
            pl.BlockSpec((R, ML), lambda i: (i, 0)),
            pl.BlockSpec((DC, DM), lambda i: (0, 0)),
            pl.BlockSpec((1, DM), lambda i: (0, 0)),
            pl.BlockSpec((MB, DM), lambda i: (0, 0)),
            pl.BlockSpec((1, DM), lambda i: (0, 0)),
            pl.BlockSpec((1, DM), lambda i: (0, 0)),
        ],
        out_specs=pl.BlockSpec((R, DM), lambda i: (i, 0)),
        out_shape=jax.ShapeDtypeStruct((NBARS, DM), jnp.float32),
    )(pooled, msk, W, b.reshape(1, DM), pos_table[:MB], gamma.reshape(1, DM),
      beta.reshape(1, DM))

    return out.reshape(B, MB, DM), bar_mask


# speedup vs baseline: 16.3280x; 16.3280x over previous
import functools

import jax
import jax.numpy as jnp
from jax import lax
from jax.experimental import pallas as pl
from jax.experimental.pallas import tpu as pltpu
from jax.experimental.pallas import tpu_sc as plsc

B, MB, ML = 1024, 64, 64
V, DC, DM = 100000, 64, 256
NBARS = B * MB
L = 16
NLC = DC // L

NC, NS = 2, 16
HALF_V = V // 2
SHARD = 50048
STAGE = SHARD // NS

BARS_PER_TILE = NBARS // NS
CB = 64
CHUNKS = BARS_PER_TILE // CB
GROUP = 2
NGRP = CB // GROUP

_PERM = ([2 * i for i in range(16)] + [2 * i + 1 for i in range(16)]
         + [32 + 2 * i for i in range(16)] + [33 + 2 * i for i in range(16)])


def _sc_pool_body(idx_hbm, msk_hbm, tbl_hbm, pooled_hbm,
                  idx_v, msk_v, idxe_v, rows0_v, rows1_v, outc_v, spm_v,
                  sem0, sem1):
    c = lax.axis_index("c")
    sid = lax.axis_index("s")
    pltpu.sync_copy(tbl_hbm.at[pl.ds(c * SHARD + sid * STAGE, STAGE)],
                    spm_v.at[pl.ds(sid * STAGE, STAGE)])
    plsc.subcore_barrier()

    vbase = c * HALF_V
    tile_bar0 = sid * BARS_PER_TILE
    rows_ring = (rows0_v, rows1_v)
    sem_ring = (sem0, sem1)

    def prep(g, s):
        for b in range(GROUP):
            for q in range(NLC):
                iv = idx_v[g * GROUP + b, pl.ds(q * L, L)]
                m = msk_v[g * GROUP + b, pl.ds(q * L, L)]
                lo = iv - vbase
                valid = (lo >= 0) & (lo < HALF_V) & (m > 0)
                idxe_v[s, pl.ds(b * ML + q * L, L)] = jnp.where(
                    valid, lo, HALF_V)

    def fire(s):
        pltpu.async_copy(spm_v.at[idxe_v.at[s]], rows_ring[s], sem_ring[s])

    def gwait(s):
        pltpu.make_async_copy(spm_v.at[idxe_v.at[s]], rows_ring[s],
                              sem_ring[s]).wait()

    def accum(g, s):
        rows = rows_ring[s]
        for b in range(GROUP):
            acc = [jnp.zeros((L,), jnp.float32) for _ in range(NLC)]
            for j in range(ML):
                for h in range(2):
                    rv = rows[b * ML + j, pl.ds(32 * h, 32)]
                    ua, ub = plsc.unpack(rv, format=plsc.PackFormat.INTERLEAVED)
                    acc[2 * h] = acc[2 * h] + ua
                    acc[2 * h + 1] = acc[2 * h + 1] + ub
            for k in range(NLC):
                outc_v[g * GROUP + b, pl.ds(k * L, L)] = acc[k]

    def chunk_body(ci, _):
        bar0 = tile_bar0 + ci * CB
        pltpu.sync_copy(idx_hbm.at[pl.ds(bar0, CB)], idx_v)
        pltpu.sync_copy(msk_hbm.at[pl.ds(bar0, CB)], msk_v)
        prep(0, 0)
        fire(0)

        def pair_body(h, _):
            g0 = 2 * h
            prep(g0 + 1, 1)
            fire(1)
            gwait(0)
            accum(g0, 0)

            @pl.when(h < NGRP // 2 - 1)
            def _():
                prep(g0 + 2, 0)
                fire(0)

            gwait(1)
            accum(g0 + 1, 1)
            return 0

        lax.fori_loop(0, NGRP // 2, pair_body, 0)
        pltpu.sync_copy(outc_v, pooled_hbm.at[c, pl.ds(bar0, CB)])
        return 0

    lax.fori_loop(0, CHUNKS, chunk_body, 0)


_sc_pool = functools.partial(
    pl.kernel,
    out_type=jax.ShapeDtypeStruct((NC, NBARS, DC), jnp.float32),
    mesh=plsc.VectorSubcoreMesh(core_axis_name="c", subcore_axis_name="s"),
    scratch_types=[
        pltpu.VMEM((CB, ML), jnp.int32),
        pltpu.VMEM((CB, ML), jnp.int32),
        pltpu.VMEM((2, GROUP * ML), jnp.int32),
        pltpu.VMEM((GROUP * ML, DC), jnp.bfloat16),
        pltpu.VMEM((GROUP * ML, DC), jnp.bfloat16),
        pltpu.VMEM((CB, DC), jnp.float32),
        pltpu.VMEM_SHARED((SHARD, DC), jnp.bfloat16),
        pltpu.SemaphoreType.DMA,
        pltpu.SemaphoreType.DMA,
    ],
    compiler_params=pltpu.CompilerParams(use_tc_tiling_on_sc=False,
                                         needs_layout_passes=False),
)(_sc_pool_body)


def _tc_body(x0_ref, x1_ref, m_ref, w_ref, b_ref, pos_ref, g_ref, bb_ref,
             o_ref):
    x = x0_ref[0] + x1_ref[0]
    cnt = jnp.sum(m_ref[...].astype(jnp.float32), axis=-1, keepdims=True)
    rinv = 1.0 / jnp.maximum(cnt, 1.0)
    y = jnp.dot(x, w_ref[...], preferred_element_type=jnp.float32)
    y = y * rinv + b_ref[...]
    r = x.shape[0] // MB
    y = (y.reshape(r, MB, DM) + pos_ref[...][None]).reshape(r * MB, DM)
    mu = jnp.mean(y, axis=-1, keepdims=True)
    d = y - mu
    var = jnp.mean(d * d, axis=-1, keepdims=True)
    o_ref[...] = d * lax.rsqrt(var + 1e-5) * g_ref[...] + bb_ref[...]


def kernel(bar_indices, char_mask, bar_mask, char_table, W, b, pos_table,
           gamma, beta):
    idx = bar_indices.astype(jnp.int32).reshape(NBARS, ML)
    msk = char_mask.astype(jnp.int32).reshape(NBARS, ML)

    tb = char_table.astype(jnp.bfloat16)
    zpad = jnp.zeros((SHARD - HALF_V, DC), jnp.bfloat16)
    tbl2 = jnp.concatenate([tb[:HALF_V], zpad, tb[HALF_V:], zpad], axis=0)

    pooled = _sc_pool(idx, msk, tbl2)

    w_perm = W[jnp.asarray(_PERM), :]

    R = 512
    out = pl.pallas_call(
        _tc_body,
        grid=(NBARS // R,),
        in_specs=[
            pl.BlockSpec((1, R, DC), lambda i: (0, i, 0)),
            pl.BlockSpec((1, R, DC), lambda i: (1, i, 0)),
            pl.BlockSpec((R, ML), lambda i: (i, 0)),
            pl.BlockSpec((DC, DM), lambda i: (0, 0)),
            pl.BlockSpec((1, DM), lambda i: (0, 0)),
            pl.BlockSpec((MB, DM), lambda i: (0, 0)),
            pl.BlockSpec((1, DM), lambda i: (0, 0)),
            pl.BlockSpec((1, DM), lambda i: (0, 0)),
        ],
        out_specs=pl.BlockSpec((R, DM), lambda i: (i, 0)),
        out_shape=jax.ShapeDtypeStruct((NBARS, DM), jnp.float32),
    )(pooled, pooled, msk, w_perm, b.reshape(1, DM), pos_table[:MB],
      gamma.reshape(1, DM), beta.reshape(1, DM))

    return out.reshape(B, MB, DM), bar_mask
